# baseline (speedup 1.0000x reference)
import functools

import jax
import jax.numpy as jnp
from jax import lax
from jax.experimental import pallas as pl
from jax.experimental.pallas import tpu as pltpu
from jax.experimental.pallas import tpu_sc as plsc

N = 10000
NPAD = 10112
E = 320000
NW = 32
CW = 64
CHUNKS = 160
NBUF = 4
EPAD = NW * CHUNKS * CW
RB = 1000
SUBROWS = NPAD // 16


def _sc_segsum(table, src_r, dst_r, zeros, zerosv, with_cnt):
    mesh = plsc.VectorSubcoreMesh(core_axis_name="c", subcore_axis_name="s")
    out_type = [jax.ShapeDtypeStruct((2, NPAD, 128), jnp.float32)]
    if with_cnt:
        out_type.append(jax.ShapeDtypeStruct((NW, NPAD), jnp.float32))

    @functools.partial(
        pl.kernel,
        out_type=tuple(out_type),
        mesh=mesh,
        compiler_params=pltpu.CompilerParams(needs_layout_passes=False),
        scratch_types=(
            [pltpu.VMEM((CW,), jnp.int32) for _ in range(2 * NBUF)]
            + [pltpu.VMEM((CW, 128), jnp.float32) for _ in range(NBUF)]
            + ([pltpu.VMEM((NPAD,), jnp.float32)] if with_cnt else [])
            + [pltpu.VMEM_SHARED((NPAD, 128), jnp.float32)]
            + [pltpu.SemaphoreType.DMA for _ in range(2 * NBUF)]
        ),
    )
    def k(table_hbm, src_hbm, dst_hbm, zeros_hbm, zerosv_hbm, *refs):
        if with_cnt:
            out_hbm, cnt_hbm = refs[0], refs[1]
            refs = refs[2:]
        else:
            out_hbm = refs[0]
            cnt_v = None
            refs = refs[1:]
        isv = refs[0:NBUF]
        idv = refs[NBUF:2 * NBUF]
        rows = refs[2 * NBUF:3 * NBUF]
        refs = refs[3 * NBUF:]
        if with_cnt:
            cnt_v = refs[0]
            refs = refs[1:]
        acc_sh = refs[0]
        gsem = refs[1:1 + NBUF]
        ssem = refs[1 + NBUF:1 + 2 * NBUF]
        cid = lax.axis_index("c")
        sid = lax.axis_index("s")
        wid = sid * 2 + cid
        r0 = sid * SUBROWS
        pltpu.sync_copy(zeros_hbm.at[pl.ds(r0, SUBROWS)],
                        acc_sh.at[pl.ds(r0, SUBROWS)])
        if with_cnt:
            pltpu.sync_copy(zerosv_hbm, cnt_v)
        plsc.subcore_barrier()
        ones16 = jnp.ones((16,), jnp.float32)
        c0 = wid * CHUNKS

        def idx_load(i, g):
            pltpu.sync_copy(src_hbm.at[c0 + g], isv[i])
            pltpu.sync_copy(dst_hbm.at[c0 + g], idv[i])

        def cnt_add(i):
            if with_cnt:
                for v in range(CW // 16):
                    plsc.addupdate_scatter(
                        cnt_v, [idv[i][pl.ds(v * 16, 16)]], ones16)

        def s_wait(i):
            pltpu.make_async_copy(rows[i], acc_sh.at[idv[i]], ssem[i]).wait()

        for i in range(NBUF):
            idx_load(i, i)
            pltpu.async_copy(table_hbm.at[isv[i]], rows[i], gsem[i])

        def body(g, carry):
            j0 = g * NBUF
            for i in range(NBUF):
                pltpu.make_async_copy(table_hbm.at[isv[i]], rows[i],
                                      gsem[i]).wait()
                pltpu.async_copy(rows[i], acc_sh.at[idv[i]], ssem[i],
                                 add=True)
                cnt_add(i)
            for i in range(NBUF):
                @pl.when(j0 + i + NBUF < CHUNKS)
                def _(i=i):
                    s_wait(i)
                    idx_load(i, j0 + i + NBUF)
                    pltpu.async_copy(table_hbm.at[isv[i]], rows[i], gsem[i])
            return carry

        lax.fori_loop(0, CHUNKS // NBUF, body, 0)
        for i in range(NBUF):
            s_wait(i)
        plsc.subcore_barrier()
        pltpu.sync_copy(acc_sh.at[pl.ds(r0, SUBROWS)],
                        out_hbm.at[cid, pl.ds(r0, SUBROWS)])
        if with_cnt:
            pltpu.sync_copy(cnt_v, cnt_hbm.at[wid])

    return k(table, src_r, dst_r, zeros, zerosv)


def _prep_edges(edge_index):
    src = edge_index[0].astype(jnp.int32)
    dst = edge_index[1].astype(jnp.int32)
    pad = EPAD - E
    src_p = jnp.concatenate([src, jnp.zeros((pad,), jnp.int32)])
    dst_p = jnp.concatenate([dst, jnp.full((pad,), N, jnp.int32)])
    return src_p.reshape(NW * CHUNKS, CW), dst_p.reshape(NW * CHUNKS, CW)


def _dotT(a, w):
    return lax.dot_general(a, w, (((1,), (1,)), ((), ())),
                           preferred_element_type=jnp.float32)


def _layer1_block(p_ref, cp_ref, x_ref, wl_ref, bl_ref, wr_ref, g_ref, b_ref,
                  xh0_ref, xh1_ref, cnt_ref):
    agg = p_ref[0] + p_ref[1]
    cnt = jnp.sum(cp_ref[...], axis=-1)
    cntc = jnp.maximum(cnt, 1.0)
    mean = agg / cntc[:, None]
    h = _dotT(mean, wl_ref[...]) + _dotT(x_ref[...], wr_ref[...]) + bl_ref[...]
    h = jnp.maximum(h, 0.0)
    mu = jnp.mean(h, axis=1, keepdims=True)
    var = jnp.mean((h - mu) ** 2, axis=1, keepdims=True)
    hn = (h - mu) * lax.rsqrt(var + 1e-5) * g_ref[...] + b_ref[...]
    xh0_ref[...] = hn[:, :128]
    xh1_ref[...] = hn[:, 128:]
    cnt_ref[...] = jnp.broadcast_to(cntc[:, None], (RB, 8))


def _layer1(P, CPT, x, Wl, bl, Wr, g, b):
    return pl.pallas_call(
        _layer1_block,
        grid=(N // RB,),
        in_specs=[
            pl.BlockSpec((2, RB, 128), lambda i: (0, i, 0)),
            pl.BlockSpec((RB, NW), lambda i: (i, 0)),
            pl.BlockSpec((RB, 128), lambda i: (i, 0)),
            pl.BlockSpec((256, 128), lambda i: (0, 0)),
            pl.BlockSpec((256,), lambda i: (0,)),
            pl.BlockSpec((256, 128), lambda i: (0, 0)),
            pl.BlockSpec((256,), lambda i: (0,)),
            pl.BlockSpec((256,), lambda i: (0,)),
        ],
        out_specs=[
            pl.BlockSpec((RB, 128), lambda i: (i, 0)),
            pl.BlockSpec((RB, 128), lambda i: (i, 0)),
            pl.BlockSpec((RB, 8), lambda i: (i, 0)),
        ],
        out_shape=[
            jax.ShapeDtypeStruct((N, 128), jnp.float32),
            jax.ShapeDtypeStruct((N, 128), jnp.float32),
            jax.ShapeDtypeStruct((N, 8), jnp.float32),
        ],
    )(P, CPT, x, Wl, bl, Wr, g, b)


def _layer2_block(qa_ref, qb_ref, cnt_ref, xh0_ref, xh1_ref, wl_ref, bl_ref,
                  wr_ref, g_ref, b_ref, w1_ref, b1_ref, w2_ref, b2_ref,
                  w3_ref, b3_ref, o_ref):
    rinv = 1.0 / cnt_ref[:, :1]
    mean = jnp.concatenate(
        [(qa_ref[0] + qa_ref[1]) * rinv, (qb_ref[0] + qb_ref[1]) * rinv],
        axis=1)
    x1 = jnp.concatenate([xh0_ref[...], xh1_ref[...]], axis=1)
    h = _dotT(mean, wl_ref[...]) + _dotT(x1, wr_ref[...]) + bl_ref[...]
    h = jnp.maximum(h, 0.0)
    mu = jnp.mean(h, axis=1, keepdims=True)
    var = jnp.mean((h - mu) ** 2, axis=1, keepdims=True)
    hn = (h - mu) * lax.rsqrt(var + 1e-5) * g_ref[...] + b_ref[...]
    z = jnp.maximum(_dotT(hn, w1_ref[...]) + b1_ref[...], 0.0)
    z = jnp.maximum(_dotT(z, w2_ref[...]) + b2_ref[...], 0.0)
    z = jnp.maximum(_dotT(z, w3_ref[...]) + b3_ref[...], 0.0)
    o_ref[...] = z


def _layer2(Qa, Qb, cnt, Xh0, Xh1, Wl, bl, Wr, g, b, W1, b1, W2, b2, W3, b3):
    full = lambda r, c: pl.BlockSpec((r, c), lambda i: (0, 0))
    vec = lambda r: pl.BlockSpec((r,), lambda i: (0,))
    return pl.pallas_call(
        _layer2_block,
        grid=(N // RB,),
        in_specs=[
            pl.BlockSpec((2, RB, 128), lambda i: (0, i, 0)),
            pl.BlockSpec((2, RB, 128), lambda i: (0, i, 0)),
            pl.BlockSpec((RB, 8), lambda i: (i, 0)),
            pl.BlockSpec((RB, 128), lambda i: (i, 0)),
            pl.BlockSpec((RB, 128), lambda i: (i, 0)),
            full(128, 256), vec(128), full(128, 256), vec(128), vec(128),
            full(256, 128), vec(256), full(128, 256), vec(128),
            full(64, 128), vec(64),
        ],
        out_specs=pl.BlockSpec((RB, 64), lambda i: (i, 0)),
        out_shape=jax.ShapeDtypeStruct((N, 64), jnp.float32),
    )(Qa, Qb, cnt, Xh0, Xh1, Wl, bl, Wr, g, b, W1, b1, W2, b2, W3, b3)


def _final_block(x_ref, y_ref, o_ref):
    o_ref[...] = lax.dot_general(x_ref[...], y_ref[...],
                                 (((1,), (1,)), ((), ())),
                                 preferred_element_type=jnp.float32)


def _final_matmul(x, y):
    RBF = 400
    return pl.pallas_call(
        _final_block,
        grid=(N // RBF,),
        in_specs=[pl.BlockSpec((RBF, 64), lambda i: (i, 0)),
                  pl.BlockSpec((N, 64), lambda i: (0, 0))],
        out_specs=pl.BlockSpec((RBF, N), lambda i: (i, 0)),
        out_shape=jax.ShapeDtypeStruct((N, N), jnp.float32),
    )(x, y)


def _branch(x, edge_index, Wl1, bl1, Wr1, g1, b1, Wl2, bl2, Wr2, g2, b2,
            W1, c1, W2, c2, W3, c3):
    src_r, dst_r = _prep_edges(edge_index)
    zeros128 = jnp.zeros((NPAD, 128), jnp.float32)
    zerosv = jnp.zeros((NPAD,), jnp.float32)
    P, CNTP = _sc_segsum(x, src_r, dst_r, zeros128, zerosv, True)
    Xh0, Xh1, cnt = _layer1(P, CNTP.T, x, Wl1, bl1, Wr1, g1, b1)
    (Qa,) = _sc_segsum(Xh0, src_r, dst_r, zeros128, zerosv, False)
    (Qb,) = _sc_segsum(Xh1, src_r, dst_r, zeros128, zerosv, False)
    return _layer2(Qa, Qb, cnt, Xh0, Xh1, Wl2, bl2, Wr2, g2, b2,
                   W1, c1, W2, c2, W3, c3)


def kernel(mm_edge_index, dd_edge_index, x_m, x_d,
           Wl_x1, bl_x1, Wr_x1, g_x1, b_x1,
           Wl_x2, bl_x2, Wr_x2, g_x2, b_x2,
           Wl_y1, bl_y1, Wr_y1, g_y1, b_y1,
           Wl_y2, bl_y2, Wr_y2, g_y2, b_y2,
           W1x, b1x, W2x, b2x, W3x, b3x,
           W1y, b1y, W2y, b2y, W3y, b3y):
    xk = _branch(x_m, mm_edge_index, Wl_x1, bl_x1, Wr_x1, g_x1, b_x1,
                 Wl_x2, bl_x2, Wr_x2, g_x2, b_x2,
                 W1x, b1x, W2x, b2x, W3x, b3x)
    yk = _branch(x_d, dd_edge_index, Wl_y1, bl_y1, Wr_y1, g_y1, b_y1,
                 Wl_y2, bl_y2, Wr_y2, g_y2, b_y2,
                 W1y, b1y, W2y, b2y, W3y, b3y)
    return _final_matmul(xk, yk)


# speedup vs baseline: 1.0724x; 1.0724x over previous
import functools

import jax
import jax.numpy as jnp
from jax import lax
from jax.experimental import pallas as pl
from jax.experimental.pallas import tpu as pltpu
from jax.experimental.pallas import tpu_sc as plsc

N = 10000
NPAD = 10112
E = 320000
NW = 32
CW = 64
CHUNKS = 160
IB = 32
NBLK = CHUNKS // IB
EPAD = NW * CHUNKS * CW
RB = 1000
SUBROWS = NPAD // 16


def _sc_segsum(table, src_r, dst_r, zeros, zerosv, with_cnt):
    mesh = plsc.VectorSubcoreMesh(core_axis_name="c", subcore_axis_name="s")
    out_type = [jax.ShapeDtypeStruct((2, NPAD, 128), jnp.float32)]
    if with_cnt:
        out_type.append(jax.ShapeDtypeStruct((NW, NPAD), jnp.float32))

    @functools.partial(
        pl.kernel,
        out_type=tuple(out_type),
        mesh=mesh,
        compiler_params=pltpu.CompilerParams(needs_layout_passes=False),
        scratch_types=(
            [pltpu.VMEM((IB, CW), jnp.int32) for _ in range(2)]
            + [pltpu.VMEM((CW, 128), jnp.float32) for _ in range(2)]
            + ([pltpu.VMEM((NPAD,), jnp.float32)] if with_cnt else [])
            + [pltpu.VMEM_SHARED((NPAD, 128), jnp.float32)]
            + [pltpu.SemaphoreType.DMA for _ in range(4)]
        ),
    )
    def k(table_hbm, src_hbm, dst_hbm, zeros_hbm, zerosv_hbm, *refs):
        if with_cnt:
            out_hbm, cnt_hbm = refs[0], refs[1]
            refs = refs[2:]
        else:
            out_hbm = refs[0]
            cnt_v = None
            refs = refs[1:]
        is_v, id_v, rows_a, rows_b = refs[0:4]
        refs = refs[4:]
        if with_cnt:
            cnt_v = refs[0]
            refs = refs[1:]
        acc_sh, gsa, gsb, ssa, ssb = refs
        cid = lax.axis_index("c")
        sid = lax.axis_index("s")
        wid = sid * 2 + cid
        r0 = sid * SUBROWS
        pltpu.sync_copy(zeros_hbm.at[pl.ds(r0, SUBROWS)],
                        acc_sh.at[pl.ds(r0, SUBROWS)])
        if with_cnt:
            pltpu.sync_copy(zerosv_hbm, cnt_v)
        plsc.subcore_barrier()
        ones16 = jnp.ones((16,), jnp.float32)
        c0 = wid * CHUNKS

        def g_issue(q, buf, sem):
            pltpu.async_copy(table_hbm.at[is_v.at[q]], buf, sem)

        def g_drain(q, buf, sem):
            pltpu.make_async_copy(table_hbm.at[is_v.at[q]], buf, sem).wait()

        def cnt_add(q):
            if with_cnt:
                for v in range(CW // 16):
                    plsc.addupdate_scatter(
                        cnt_v, [id_v[q, pl.ds(v * 16, 16)]], ones16)

        def blk_body(blk, carry):
            b0 = c0 + blk * IB
            pltpu.sync_copy(src_hbm.at[pl.ds(b0, IB)], is_v)
            pltpu.sync_copy(dst_hbm.at[pl.ds(b0, IB)], id_v)
            g_issue(0, rows_a, gsa)
            g_issue(1, rows_b, gsb)

            def pair(p, carry2):
                qa = 2 * p
                qb = 2 * p + 1
                g_drain(qa, rows_a, gsa)
                da = pltpu.async_copy(rows_a, acc_sh.at[id_v.at[qa]], ssa,
                                      add=True)
                cnt_add(qa)
                da.wait()

                @pl.when(qa + 2 < IB)
                def _():
                    g_issue(qa + 2, rows_a, gsa)

                g_drain(qb, rows_b, gsb)
                db = pltpu.async_copy(rows_b, acc_sh.at[id_v.at[qb]], ssb,
                                      add=True)
                cnt_add(qb)
                db.wait()

                @pl.when(qb + 2 < IB)
                def _():
                    g_issue(qb + 2, rows_b, gsb)

                return carry2

            lax.fori_loop(0, IB // 2, pair, 0)
            return carry

        lax.fori_loop(0, NBLK, blk_body, 0)
        plsc.subcore_barrier()
        pltpu.sync_copy(acc_sh.at[pl.ds(r0, SUBROWS)],
                        out_hbm.at[cid, pl.ds(r0, SUBROWS)])
        if with_cnt:
            pltpu.sync_copy(cnt_v, cnt_hbm.at[wid])

    return k(table, src_r, dst_r, zeros, zerosv)


def _prep_edges(edge_index):
    src = edge_index[0].astype(jnp.int32)
    dst = edge_index[1].astype(jnp.int32)
    pad = EPAD - E
    src_p = jnp.concatenate([src, jnp.zeros((pad,), jnp.int32)])
    dst_p = jnp.concatenate([dst, jnp.full((pad,), N, jnp.int32)])
    return src_p.reshape(NW * CHUNKS, CW), dst_p.reshape(NW * CHUNKS, CW)


def _dotT(a, w):
    return lax.dot_general(a, w, (((1,), (1,)), ((), ())),
                           preferred_element_type=jnp.float32)


def _layer1_block(p_ref, cp_ref, x_ref, wl_ref, bl_ref, wr_ref, g_ref, b_ref,
                  xh0_ref, xh1_ref, cnt_ref):
    agg = p_ref[0] + p_ref[1]
    cnt = jnp.sum(cp_ref[...], axis=-1)
    cntc = jnp.maximum(cnt, 1.0)
    mean = agg / cntc[:, None]
    h = _dotT(mean, wl_ref[...]) + _dotT(x_ref[...], wr_ref[...]) + bl_ref[...]
    h = jnp.maximum(h, 0.0)
    mu = jnp.mean(h, axis=1, keepdims=True)
    var = jnp.mean((h - mu) ** 2, axis=1, keepdims=True)
    hn = (h - mu) * lax.rsqrt(var + 1e-5) * g_ref[...] + b_ref[...]
    xh0_ref[...] = hn[:, :128]
    xh1_ref[...] = hn[:, 128:]
    cnt_ref[...] = jnp.broadcast_to(cntc[:, None], (RB, 8))


def _layer1(P, CPT, x, Wl, bl, Wr, g, b):
    return pl.pallas_call(
        _layer1_block,
        grid=(N // RB,),
        in_specs=[
            pl.BlockSpec((2, RB, 128), lambda i: (0, i, 0)),
            pl.BlockSpec((RB, NW), lambda i: (i, 0)),
            pl.BlockSpec((RB, 128), lambda i: (i, 0)),
            pl.BlockSpec((256, 128), lambda i: (0, 0)),
            pl.BlockSpec((256,), lambda i: (0,)),
            pl.BlockSpec((256, 128), lambda i: (0, 0)),
            pl.BlockSpec((256,), lambda i: (0,)),
            pl.BlockSpec((256,), lambda i: (0,)),
        ],
        out_specs=[
            pl.BlockSpec((RB, 128), lambda i: (i, 0)),
            pl.BlockSpec((RB, 128), lambda i: (i, 0)),
            pl.BlockSpec((RB, 8), lambda i: (i, 0)),
        ],
        out_shape=[
            jax.ShapeDtypeStruct((N, 128), jnp.float32),
            jax.ShapeDtypeStruct((N, 128), jnp.float32),
            jax.ShapeDtypeStruct((N, 8), jnp.float32),
        ],
    )(P, CPT, x, Wl, bl, Wr, g, b)


def _layer2_block(qa_ref, qb_ref, cnt_ref, xh0_ref, xh1_ref, wl_ref, bl_ref,
                  wr_ref, g_ref, b_ref, w1_ref, b1_ref, w2_ref, b2_ref,
                  w3_ref, b3_ref, o_ref):
    rinv = 1.0 / cnt_ref[:, :1]
    mean = jnp.concatenate(
        [(qa_ref[0] + qa_ref[1]) * rinv, (qb_ref[0] + qb_ref[1]) * rinv],
        axis=1)
    x1 = jnp.concatenate([xh0_ref[...], xh1_ref[...]], axis=1)
    h = _dotT(mean, wl_ref[...]) + _dotT(x1, wr_ref[...]) + bl_ref[...]
    h = jnp.maximum(h, 0.0)
    mu = jnp.mean(h, axis=1, keepdims=True)
    var = jnp.mean((h - mu) ** 2, axis=1, keepdims=True)
    hn = (h - mu) * lax.rsqrt(var + 1e-5) * g_ref[...] + b_ref[...]
    z = jnp.maximum(_dotT(hn, w1_ref[...]) + b1_ref[...], 0.0)
    z = jnp.maximum(_dotT(z, w2_ref[...]) + b2_ref[...], 0.0)
    z = jnp.maximum(_dotT(z, w3_ref[...]) + b3_ref[...], 0.0)
    o_ref[...] = z


def _layer2(Qa, Qb, cnt, Xh0, Xh1, Wl, bl, Wr, g, b, W1, b1, W2, b2, W3, b3):
    full = lambda r, c: pl.BlockSpec((r, c), lambda i: (0, 0))
    vec = lambda r: pl.BlockSpec((r,), lambda i: (0,))
    return pl.pallas_call(
        _layer2_block,
        grid=(N // RB,),
        in_specs=[
            pl.BlockSpec((2, RB, 128), lambda i: (0, i, 0)),
            pl.BlockSpec((2, RB, 128), lambda i: (0, i, 0)),
            pl.BlockSpec((RB, 8), lambda i: (i, 0)),
            pl.BlockSpec((RB, 128), lambda i: (i, 0)),
            pl.BlockSpec((RB, 128), lambda i: (i, 0)),
            full(128, 256), vec(128), full(128, 256), vec(128), vec(128),
            full(256, 128), vec(256), full(128, 256), vec(128),
            full(64, 128), vec(64),
        ],
        out_specs=pl.BlockSpec((RB, 64), lambda i: (i, 0)),
        out_shape=jax.ShapeDtypeStruct((N, 64), jnp.float32),
    )(Qa, Qb, cnt, Xh0, Xh1, Wl, bl, Wr, g, b, W1, b1, W2, b2, W3, b3)


def _final_block(x_ref, y_ref, o_ref):
    o_ref[...] = lax.dot_general(x_ref[...], y_ref[...],
                                 (((1,), (1,)), ((), ())),
                                 preferred_element_type=jnp.float32)


def _final_matmul(x, y):
    RBF = 400
    return pl.pallas_call(
        _final_block,
        grid=(N // RBF,),
        in_specs=[pl.BlockSpec((RBF, 64), lambda i: (i, 0)),
                  pl.BlockSpec((N, 64), lambda i: (0, 0))],
        out_specs=pl.BlockSpec((RBF, N), lambda i: (i, 0)),
        out_shape=jax.ShapeDtypeStruct((N, N), jnp.float32),
    )(x, y)


def _branch(x, edge_index, Wl1, bl1, Wr1, g1, b1, Wl2, bl2, Wr2, g2, b2,
            W1, c1, W2, c2, W3, c3):
    src_r, dst_r = _prep_edges(edge_index)
    zeros128 = jnp.zeros((NPAD, 128), jnp.float32)
    zerosv = jnp.zeros((NPAD,), jnp.float32)
    P, CNTP = _sc_segsum(x, src_r, dst_r, zeros128, zerosv, True)
    Xh0, Xh1, cnt = _layer1(P, CNTP.T, x, Wl1, bl1, Wr1, g1, b1)
    (Qa,) = _sc_segsum(Xh0, src_r, dst_r, zeros128, zerosv, False)
    (Qb,) = _sc_segsum(Xh1, src_r, dst_r, zeros128, zerosv, False)
    return _layer2(Qa, Qb, cnt, Xh0, Xh1, Wl2, bl2, Wr2, g2, b2,
                   W1, c1, W2, c2, W3, c3)


def kernel(mm_edge_index, dd_edge_index, x_m, x_d,
           Wl_x1, bl_x1, Wr_x1, g_x1, b_x1,
           Wl_x2, bl_x2, Wr_x2, g_x2, b_x2,
           Wl_y1, bl_y1, Wr_y1, g_y1, b_y1,
           Wl_y2, bl_y2, Wr_y2, g_y2, b_y2,
           W1x, b1x, W2x, b2x, W3x, b3x,
           W1y, b1y, W2y, b2y, W3y, b3y):
    xk = _branch(x_m, mm_edge_index, Wl_x1, bl_x1, Wr_x1, g_x1, b_x1,
                 Wl_x2, bl_x2, Wr_x2, g_x2, b_x2,
                 W1x, b1x, W2x, b2x, W3x, b3x)
    yk = _branch(x_d, dd_edge_index, Wl_y1, bl_y1, Wr_y1, g_y1, b_y1,
                 Wl_y2, bl_y2, Wr_y2, g_y2, b_y2,
                 W1y, b1y, W2y, b2y, W3y, b3y)
    return _final_matmul(xk, yk)


# speedup vs baseline: 1.4421x; 1.3447x over previous
import functools

import jax
import jax.numpy as jnp
from jax import lax
from jax.experimental import pallas as pl
from jax.experimental.pallas import tpu as pltpu
from jax.experimental.pallas import tpu_sc as plsc

N = 10000
NPAD = 10112
E = 320000
NW = 32
CW = 64
CHUNKS = 160
IB = 32
NBLK = CHUNKS // IB
EPAD = NW * CHUNKS * CW
RB = 1000
SUBROWS = NPAD // 16


def _sc_segsum(table, src_r, dst_r, zeros, zerosv, with_cnt):
    mesh = plsc.VectorSubcoreMesh(core_axis_name="c", subcore_axis_name="s")
    out_type = [jax.ShapeDtypeStruct((2, NPAD, 128), jnp.float32)]
    if with_cnt:
        out_type.append(jax.ShapeDtypeStruct((NW, NPAD), jnp.float32))

    @functools.partial(
        pl.kernel,
        out_type=tuple(out_type),
        mesh=mesh,
        compiler_params=pltpu.CompilerParams(needs_layout_passes=False),
        scratch_types=(
            [pltpu.VMEM((IB, CW), jnp.int32) for _ in range(2)]
            + [pltpu.VMEM((CW, 128), jnp.float32) for _ in range(2)]
            + ([pltpu.VMEM((NPAD,), jnp.float32)] if with_cnt else [])
            + [pltpu.VMEM_SHARED((NPAD, 128), jnp.float32)]
            + [pltpu.SemaphoreType.DMA for _ in range(4)]
        ),
    )
    def k(table_hbm, src_hbm, dst_hbm, zeros_hbm, zerosv_hbm, *refs):
        if with_cnt:
            out_hbm, cnt_hbm = refs[0], refs[1]
            refs = refs[2:]
        else:
            out_hbm = refs[0]
            cnt_v = None
            refs = refs[1:]
        is_v, id_v, rows_a, rows_b = refs[0:4]
        refs = refs[4:]
        if with_cnt:
            cnt_v = refs[0]
            refs = refs[1:]
        acc_sh, gsa, gsb, ssa, ssb = refs
        cid = lax.axis_index("c")
        sid = lax.axis_index("s")
        wid = sid * 2 + cid
        r0 = sid * SUBROWS
        pltpu.sync_copy(zeros_hbm.at[pl.ds(r0, SUBROWS)],
                        acc_sh.at[pl.ds(r0, SUBROWS)])
        if with_cnt:
            pltpu.sync_copy(zerosv_hbm, cnt_v)
        plsc.subcore_barrier()
        ones16 = jnp.ones((16,), jnp.float32)
        c0 = wid * CHUNKS

        def g_issue(q, buf, sem):
            pltpu.async_copy(table_hbm.at[is_v.at[q]], buf, sem)

        def g_drain(q, buf, sem):
            pltpu.make_async_copy(table_hbm.at[is_v.at[q]], buf, sem).wait()

        def cnt_add(q):
            if with_cnt:
                for v in range(CW // 16):
                    plsc.addupdate_scatter(
                        cnt_v, [id_v[q, pl.ds(v * 16, 16)]], ones16)

        def blk_body(blk, carry):
            b0 = c0 + blk * IB
            pltpu.sync_copy(src_hbm.at[pl.ds(b0, IB)], is_v)
            pltpu.sync_copy(dst_hbm.at[pl.ds(b0, IB)], id_v)
            g_issue(0, rows_a, gsa)
            g_issue(1, rows_b, gsb)

            def pair(p, carry2):
                qa = 2 * p
                qb = 2 * p + 1
                g_drain(qa, rows_a, gsa)
                da = pltpu.async_copy(rows_a, acc_sh.at[id_v.at[qa]], ssa,
                                      add=True)
                cnt_add(qa)
                da.wait()

                @pl.when(qa + 2 < IB)
                def _():
                    g_issue(qa + 2, rows_a, gsa)

                g_drain(qb, rows_b, gsb)
                db = pltpu.async_copy(rows_b, acc_sh.at[id_v.at[qb]], ssb,
                                      add=True)
                cnt_add(qb)
                db.wait()

                @pl.when(qb + 2 < IB)
                def _():
                    g_issue(qb + 2, rows_b, gsb)

                return carry2

            lax.fori_loop(0, IB // 2, pair, 0)
            return carry

        lax.fori_loop(0, NBLK, blk_body, 0)
        plsc.subcore_barrier()
        pltpu.sync_copy(acc_sh.at[pl.ds(r0, SUBROWS)],
                        out_hbm.at[cid, pl.ds(r0, SUBROWS)])
        if with_cnt:
            pltpu.sync_copy(cnt_v, cnt_hbm.at[wid])

    return k(table, src_r, dst_r, zeros, zerosv)


def _prep_edges(edge_index):
    src = edge_index[0].astype(jnp.int32)
    dst = edge_index[1].astype(jnp.int32)
    pad = EPAD - E
    src_p = jnp.concatenate([src, jnp.zeros((pad,), jnp.int32)])
    dst_p = jnp.concatenate([dst, jnp.full((pad,), N, jnp.int32)])
    return src_p.reshape(NW * CHUNKS, CW), dst_p.reshape(NW * CHUNKS, CW)


def _dotT(a, w):
    return lax.dot_general(a, w, (((1,), (1,)), ((), ())),
                           preferred_element_type=jnp.float32)


def _layer1_block(p_ref, cp_ref, x_ref, wl_ref, bl_ref, wr_ref, g_ref, b_ref,
                  wl2_ref, wr2_ref, xp_ref, xr_ref, cnt_ref):
    agg = p_ref[0] + p_ref[1]
    cnt = jnp.sum(cp_ref[...], axis=-1)
    cntc = jnp.maximum(cnt, 1.0)
    mean = agg / cntc[:, None]
    h = _dotT(mean, wl_ref[...]) + _dotT(x_ref[...], wr_ref[...]) + bl_ref[...]
    h = jnp.maximum(h, 0.0)
    mu = jnp.mean(h, axis=1, keepdims=True)
    var = jnp.mean((h - mu) ** 2, axis=1, keepdims=True)
    hn = (h - mu) * lax.rsqrt(var + 1e-5) * g_ref[...] + b_ref[...]
    xp_ref[...] = _dotT(hn, wl2_ref[...])
    xr_ref[...] = _dotT(hn, wr2_ref[...])
    cnt_ref[...] = jnp.broadcast_to(cntc[:, None], (RB, 8))


def _layer1(P, CPT, x, Wl, bl, Wr, g, b, Wl2, Wr2):
    return pl.pallas_call(
        _layer1_block,
        grid=(N // RB,),
        in_specs=[
            pl.BlockSpec((2, RB, 128), lambda i: (0, i, 0)),
            pl.BlockSpec((RB, NW), lambda i: (i, 0)),
            pl.BlockSpec((RB, 128), lambda i: (i, 0)),
            pl.BlockSpec((256, 128), lambda i: (0, 0)),
            pl.BlockSpec((256,), lambda i: (0,)),
            pl.BlockSpec((256, 128), lambda i: (0, 0)),
            pl.BlockSpec((256,), lambda i: (0,)),
            pl.BlockSpec((256,), lambda i: (0,)),
            pl.BlockSpec((128, 256), lambda i: (0, 0)),
            pl.BlockSpec((128, 256), lambda i: (0, 0)),
        ],
        out_specs=[
            pl.BlockSpec((RB, 128), lambda i: (i, 0)),
            pl.BlockSpec((RB, 128), lambda i: (i, 0)),
            pl.BlockSpec((RB, 8), lambda i: (i, 0)),
        ],
        out_shape=[
            jax.ShapeDtypeStruct((N, 128), jnp.float32),
            jax.ShapeDtypeStruct((N, 128), jnp.float32),
            jax.ShapeDtypeStruct((N, 8), jnp.float32),
        ],
    )(P, CPT, x, Wl, bl, Wr, g, b, Wl2, Wr2)


def _layer2_block(q_ref, cnt_ref, xr_ref, bl_ref, g_ref, b_ref,
                  w1_ref, b1_ref, w2_ref, b2_ref, w3_ref, b3_ref, o_ref):
    rinv = 1.0 / cnt_ref[:, :1]
    h = (q_ref[0] + q_ref[1]) * rinv + xr_ref[...] + bl_ref[...]
    h = jnp.maximum(h, 0.0)
    mu = jnp.mean(h, axis=1, keepdims=True)
    var = jnp.mean((h - mu) ** 2, axis=1, keepdims=True)
    hn = (h - mu) * lax.rsqrt(var + 1e-5) * g_ref[...] + b_ref[...]
    z = jnp.maximum(_dotT(hn, w1_ref[...]) + b1_ref[...], 0.0)
    z = jnp.maximum(_dotT(z, w2_ref[...]) + b2_ref[...], 0.0)
    z = jnp.maximum(_dotT(z, w3_ref[...]) + b3_ref[...], 0.0)
    o_ref[...] = z


def _layer2(Q, cnt, Xr, bl, g, b, W1, b1, W2, b2, W3, b3):
    full = lambda r, c: pl.BlockSpec((r, c), lambda i: (0, 0))
    vec = lambda r: pl.BlockSpec((r,), lambda i: (0,))
    return pl.pallas_call(
        _layer2_block,
        grid=(N // RB,),
        in_specs=[
            pl.BlockSpec((2, RB, 128), lambda i: (0, i, 0)),
            pl.BlockSpec((RB, 8), lambda i: (i, 0)),
            pl.BlockSpec((RB, 128), lambda i: (i, 0)),
            vec(128), vec(128), vec(128),
            full(256, 128), vec(256), full(128, 256), vec(128),
            full(64, 128), vec(64),
        ],
        out_specs=pl.BlockSpec((RB, 64), lambda i: (i, 0)),
        out_shape=jax.ShapeDtypeStruct((N, 64), jnp.float32),
    )(Q, cnt, Xr, bl, g, b, W1, b1, W2, b2, W3, b3)


def _final_block(x_ref, y_ref, o_ref):
    o_ref[...] = lax.dot_general(x_ref[...], y_ref[...],
                                 (((1,), (1,)), ((), ())),
                                 preferred_element_type=jnp.float32)


def _final_matmul(x, y):
    RBF = 400
    return pl.pallas_call(
        _final_block,
        grid=(N // RBF,),
        in_specs=[pl.BlockSpec((RBF, 64), lambda i: (i, 0)),
                  pl.BlockSpec((N, 64), lambda i: (0, 0))],
        out_specs=pl.BlockSpec((RBF, N), lambda i: (i, 0)),
        out_shape=jax.ShapeDtypeStruct((N, N), jnp.float32),
    )(x, y)


def _branch(x, edge_index, Wl1, bl1, Wr1, g1, b1, Wl2, bl2, Wr2, g2, b2,
            W1, c1, W2, c2, W3, c3):
    src_r, dst_r = _prep_edges(edge_index)
    zeros128 = jnp.zeros((NPAD, 128), jnp.float32)
    zerosv = jnp.zeros((NPAD,), jnp.float32)
    P, CNTP = _sc_segsum(x, src_r, dst_r, zeros128, zerosv, True)
    Xp, Xr, cnt = _layer1(P, CNTP.T, x, Wl1, bl1, Wr1, g1, b1, Wl2, Wr2)
    (Q,) = _sc_segsum(Xp, src_r, dst_r, zeros128, zerosv, False)
    return _layer2(Q, cnt, Xr, bl2, g2, b2, W1, c1, W2, c2, W3, c3)


def kernel(mm_edge_index, dd_edge_index, x_m, x_d,
           Wl_x1, bl_x1, Wr_x1, g_x1, b_x1,
           Wl_x2, bl_x2, Wr_x2, g_x2, b_x2,
           Wl_y1, bl_y1, Wr_y1, g_y1, b_y1,
           Wl_y2, bl_y2, Wr_y2, g_y2, b_y2,
           W1x, b1x, W2x, b2x, W3x, b3x,
           W1y, b1y, W2y, b2y, W3y, b3y):
    xk = _branch(x_m, mm_edge_index, Wl_x1, bl_x1, Wr_x1, g_x1, b_x1,
                 Wl_x2, bl_x2, Wr_x2, g_x2, b_x2,
                 W1x, b1x, W2x, b2x, W3x, b3x)
    yk = _branch(x_d, dd_edge_index, Wl_y1, bl_y1, Wr_y1, g_y1, b_y1,
                 Wl_y2, bl_y2, Wr_y2, g_y2, b_y2,
                 W1y, b1y, W2y, b2y, W3y, b3y)
    return _final_matmul(xk, yk)


# speedup vs baseline: 1.5012x; 1.0410x over previous
import functools

import jax
import jax.numpy as jnp
from jax import lax
from jax.experimental import pallas as pl
from jax.experimental.pallas import tpu as pltpu
from jax.experimental.pallas import tpu_sc as plsc

N = 10000
NPAD = 10112
E = 320000
NW = 32
CW = 64
CW2 = 128
EPAD = 327680
RB = 1000
SUBROWS = NPAD // 16


def _sc_segsum(table, src_r, dst_r, zeros, zerosv, with_cnt, cw=CW):
    chunks = EPAD // (NW * cw)
    ib = 32 if cw <= 64 else 16
    nblk = chunks // ib
    mesh = plsc.VectorSubcoreMesh(core_axis_name="c", subcore_axis_name="s")
    out_type = [jax.ShapeDtypeStruct((2, NPAD, 128), jnp.float32)]
    if with_cnt:
        out_type.append(jax.ShapeDtypeStruct((NW, NPAD), jnp.float32))

    @functools.partial(
        pl.kernel,
        out_type=tuple(out_type),
        mesh=mesh,
        compiler_params=pltpu.CompilerParams(needs_layout_passes=False),
        scratch_types=(
            [pltpu.VMEM((ib, cw), jnp.int32) for _ in range(2)]
            + [pltpu.VMEM((cw, 128), jnp.float32) for _ in range(2)]
            + ([pltpu.VMEM((NPAD,), jnp.float32)] if with_cnt else [])
            + [pltpu.VMEM_SHARED((NPAD, 128), jnp.float32)]
            + [pltpu.SemaphoreType.DMA for _ in range(4)]
        ),
    )
    def k(table_hbm, src_hbm, dst_hbm, zeros_hbm, zerosv_hbm, *refs):
        if with_cnt:
            out_hbm, cnt_hbm = refs[0], refs[1]
            refs = refs[2:]
        else:
            out_hbm = refs[0]
            cnt_v = None
            refs = refs[1:]
        is_v, id_v, rows_a, rows_b = refs[0:4]
        refs = refs[4:]
        if with_cnt:
            cnt_v = refs[0]
            refs = refs[1:]
        acc_sh, gsa, gsb, ssa, ssb = refs
        cid = lax.axis_index("c")
        sid = lax.axis_index("s")
        wid = sid * 2 + cid
        r0 = sid * SUBROWS
        pltpu.sync_copy(zeros_hbm.at[pl.ds(r0, SUBROWS)],
                        acc_sh.at[pl.ds(r0, SUBROWS)])
        if with_cnt:
            pltpu.sync_copy(zerosv_hbm, cnt_v)
        plsc.subcore_barrier()
        ones16 = jnp.ones((16,), jnp.float32)
        c0 = wid * chunks

        def g_issue(q, buf, sem):
            pltpu.async_copy(table_hbm.at[is_v.at[q]], buf, sem)

        def g_drain(q, buf, sem):
            pltpu.make_async_copy(table_hbm.at[is_v.at[q]], buf, sem).wait()

        def cnt_add(q):
            if with_cnt:
                for v in range(cw // 16):
                    plsc.addupdate_scatter(
                        cnt_v, [id_v[q, pl.ds(v * 16, 16)]], ones16)

        def blk_body(blk, carry):
            b0 = c0 + blk * ib
            pltpu.sync_copy(src_hbm.at[pl.ds(b0, ib)], is_v)
            pltpu.sync_copy(dst_hbm.at[pl.ds(b0, ib)], id_v)
            g_issue(0, rows_a, gsa)
            g_issue(1, rows_b, gsb)

            def pair(p, carry2):
                qa = 2 * p
                qb = 2 * p + 1
                g_drain(qa, rows_a, gsa)
                da = pltpu.async_copy(rows_a, acc_sh.at[id_v.at[qa]], ssa,
                                      add=True)
                cnt_add(qa)
                da.wait()

                @pl.when(qa + 2 < ib)
                def _():
                    g_issue(qa + 2, rows_a, gsa)

                g_drain(qb, rows_b, gsb)
                db = pltpu.async_copy(rows_b, acc_sh.at[id_v.at[qb]], ssb,
                                      add=True)
                cnt_add(qb)
                db.wait()

                @pl.when(qb + 2 < ib)
                def _():
                    g_issue(qb + 2, rows_b, gsb)

                return carry2

            lax.fori_loop(0, ib // 2, pair, 0)
            return carry

        lax.fori_loop(0, nblk, blk_body, 0)
        plsc.subcore_barrier()
        pltpu.sync_copy(acc_sh.at[pl.ds(r0, SUBROWS)],
                        out_hbm.at[cid, pl.ds(r0, SUBROWS)])
        if with_cnt:
            pltpu.sync_copy(cnt_v, cnt_hbm.at[wid])

    return k(table, src_r, dst_r, zeros, zerosv)


def _prep_edges(edge_index, cw):
    src = edge_index[0].astype(jnp.int32)
    dst = edge_index[1].astype(jnp.int32)
    pad = EPAD - E
    src_p = jnp.concatenate([src, jnp.zeros((pad,), jnp.int32)])
    dst_p = jnp.concatenate([dst, jnp.full((pad,), N, jnp.int32)])
    return src_p.reshape(EPAD // cw, cw), dst_p.reshape(EPAD // cw, cw)


def _dotT(a, w):
    return lax.dot_general(a, w, (((1,), (1,)), ((), ())),
                           preferred_element_type=jnp.float32)


def _layer1_block(p_ref, cp_ref, x_ref, wl_ref, bl_ref, wr_ref, g_ref, b_ref,
                  wl2_ref, wr2_ref, xp_ref, xr_ref, cnt_ref):
    agg = p_ref[0] + p_ref[1]
    cnt = jnp.sum(cp_ref[...], axis=-1)
    cntc = jnp.maximum(cnt, 1.0)
    mean = agg / cntc[:, None]
    h = _dotT(mean, wl_ref[...]) + _dotT(x_ref[...], wr_ref[...]) + bl_ref[...]
    h = jnp.maximum(h, 0.0)
    mu = jnp.mean(h, axis=1, keepdims=True)
    var = jnp.mean((h - mu) ** 2, axis=1, keepdims=True)
    hn = (h - mu) * lax.rsqrt(var + 1e-5) * g_ref[...] + b_ref[...]
    xp_ref[...] = _dotT(hn, wl2_ref[...])
    xr_ref[...] = _dotT(hn, wr2_ref[...])
    cnt_ref[...] = jnp.broadcast_to(cntc[:, None], (RB, 8))


def _layer1(P, CPT, x, Wl, bl, Wr, g, b, Wl2, Wr2):
    return pl.pallas_call(
        _layer1_block,
        grid=(N // RB,),
        in_specs=[
            pl.BlockSpec((2, RB, 128), lambda i: (0, i, 0)),
            pl.BlockSpec((RB, NW), lambda i: (i, 0)),
            pl.BlockSpec((RB, 128), lambda i: (i, 0)),
            pl.BlockSpec((256, 128), lambda i: (0, 0)),
            pl.BlockSpec((256,), lambda i: (0,)),
            pl.BlockSpec((256, 128), lambda i: (0, 0)),
            pl.BlockSpec((256,), lambda i: (0,)),
            pl.BlockSpec((256,), lambda i: (0,)),
            pl.BlockSpec((128, 256), lambda i: (0, 0)),
            pl.BlockSpec((128, 256), lambda i: (0, 0)),
        ],
        out_specs=[
            pl.BlockSpec((RB, 128), lambda i: (i, 0)),
            pl.BlockSpec((RB, 128), lambda i: (i, 0)),
            pl.BlockSpec((RB, 8), lambda i: (i, 0)),
        ],
        out_shape=[
            jax.ShapeDtypeStruct((N, 128), jnp.float32),
            jax.ShapeDtypeStruct((N, 128), jnp.float32),
            jax.ShapeDtypeStruct((N, 8), jnp.float32),
        ],
    )(P, CPT, x, Wl, bl, Wr, g, b, Wl2, Wr2)


def _layer2_block(q_ref, cnt_ref, xr_ref, bl_ref, g_ref, b_ref,
                  w1_ref, b1_ref, w2_ref, b2_ref, w3_ref, b3_ref, o_ref):
    rinv = 1.0 / cnt_ref[:, :1]
    h = (q_ref[0] + q_ref[1]) * rinv + xr_ref[...] + bl_ref[...]
    h = jnp.maximum(h, 0.0)
    mu = jnp.mean(h, axis=1, keepdims=True)
    var = jnp.mean((h - mu) ** 2, axis=1, keepdims=True)
    hn = (h - mu) * lax.rsqrt(var + 1e-5) * g_ref[...] + b_ref[...]
    z = jnp.maximum(_dotT(hn, w1_ref[...]) + b1_ref[...], 0.0)
    z = jnp.maximum(_dotT(z, w2_ref[...]) + b2_ref[...], 0.0)
    z = jnp.maximum(_dotT(z, w3_ref[...]) + b3_ref[...], 0.0)
    o_ref[...] = z


def _layer2(Q, cnt, Xr, bl, g, b, W1, b1, W2, b2, W3, b3):
    full = lambda r, c: pl.BlockSpec((r, c), lambda i: (0, 0))
    vec = lambda r: pl.BlockSpec((r,), lambda i: (0,))
    return pl.pallas_call(
        _layer2_block,
        grid=(N // RB,),
        in_specs=[
            pl.BlockSpec((2, RB, 128), lambda i: (0, i, 0)),
            pl.BlockSpec((RB, 8), lambda i: (i, 0)),
            pl.BlockSpec((RB, 128), lambda i: (i, 0)),
            vec(128), vec(128), vec(128),
            full(256, 128), vec(256), full(128, 256), vec(128),
            full(64, 128), vec(64),
        ],
        out_specs=pl.BlockSpec((RB, 64), lambda i: (i, 0)),
        out_shape=jax.ShapeDtypeStruct((N, 64), jnp.float32),
    )(Q, cnt, Xr, bl, g, b, W1, b1, W2, b2, W3, b3)


def _final_block(x_ref, y_ref, o_ref):
    o_ref[...] = lax.dot_general(x_ref[...], y_ref[...],
                                 (((1,), (1,)), ((), ())),
                                 preferred_element_type=jnp.float32)


def _final_matmul(x, y):
    RBF = 400
    return pl.pallas_call(
        _final_block,
        grid=(N // RBF,),
        in_specs=[pl.BlockSpec((RBF, 64), lambda i: (i, 0)),
                  pl.BlockSpec((N, 64), lambda i: (0, 0))],
        out_specs=pl.BlockSpec((RBF, N), lambda i: (i, 0)),
        out_shape=jax.ShapeDtypeStruct((N, N), jnp.float32),
    )(x, y)


def _branch(x, edge_index, Wl1, bl1, Wr1, g1, b1, Wl2, bl2, Wr2, g2, b2,
            W1, c1, W2, c2, W3, c3):
    src_r, dst_r = _prep_edges(edge_index, CW)
    src_r2, dst_r2 = _prep_edges(edge_index, CW2)
    zeros128 = jnp.zeros((NPAD, 128), jnp.float32)
    zerosv = jnp.zeros((NPAD,), jnp.float32)
    P, CNTP = _sc_segsum(x, src_r, dst_r, zeros128, zerosv, True, CW)
    Xp, Xr, cnt = _layer1(P, CNTP.T, x, Wl1, bl1, Wr1, g1, b1, Wl2, Wr2)
    (Q,) = _sc_segsum(Xp, src_r2, dst_r2, zeros128, zerosv, False, CW2)
    return _layer2(Q, cnt, Xr, bl2, g2, b2, W1, c1, W2, c2, W3, c3)


def kernel(mm_edge_index, dd_edge_index, x_m, x_d,
           Wl_x1, bl_x1, Wr_x1, g_x1, b_x1,
           Wl_x2, bl_x2, Wr_x2, g_x2, b_x2,
           Wl_y1, bl_y1, Wr_y1, g_y1, b_y1,
           Wl_y2, bl_y2, Wr_y2, g_y2, b_y2,
           W1x, b1x, W2x, b2x, W3x, b3x,
           W1y, b1y, W2y, b2y, W3y, b3y):
    xk = _branch(x_m, mm_edge_index, Wl_x1, bl_x1, Wr_x1, g_x1, b_x1,
                 Wl_x2, bl_x2, Wr_x2, g_x2, b_x2,
                 W1x, b1x, W2x, b2x, W3x, b3x)
    yk = _branch(x_d, dd_edge_index, Wl_y1, bl_y1, Wr_y1, g_y1, b_y1,
                 Wl_y2, bl_y2, Wr_y2, g_y2, b_y2,
                 W1y, b1y, W2y, b2y, W3y, b3y)
    return _final_matmul(xk, yk)


# speedup vs baseline: 1.5152x; 1.0093x over previous
import functools

import jax
import jax.numpy as jnp
from jax import lax
from jax.experimental import pallas as pl
from jax.experimental.pallas import tpu as pltpu
from jax.experimental.pallas import tpu_sc as plsc

N = 10000
NPAD = 10112
E = 320000
NW = 32
CW = 128
CW2 = 128
EPAD = 327680
RB = 1000
SUBROWS = NPAD // 16


def _sc_segsum(table, src_r, dst_r, zeros, zerosv, with_cnt, cw=CW):
    chunks = EPAD // (NW * cw)
    ib = 32 if cw <= 64 else 16
    nblk = chunks // ib
    mesh = plsc.VectorSubcoreMesh(core_axis_name="c", subcore_axis_name="s")
    out_type = [jax.ShapeDtypeStruct((2, NPAD, 128), jnp.float32)]
    if with_cnt:
        out_type.append(jax.ShapeDtypeStruct((NW, NPAD), jnp.float32))

    @functools.partial(
        pl.kernel,
        out_type=tuple(out_type),
        mesh=mesh,
        compiler_params=pltpu.CompilerParams(needs_layout_passes=False),
        scratch_types=(
            [pltpu.VMEM((ib, cw), jnp.int32) for _ in range(2)]
            + [pltpu.VMEM((cw, 128), jnp.float32) for _ in range(2)]
            + ([pltpu.VMEM((NPAD,), jnp.float32)] if with_cnt else [])
            + [pltpu.VMEM_SHARED((NPAD, 128), jnp.float32)]
            + [pltpu.SemaphoreType.DMA for _ in range(4)]
        ),
    )
    def k(table_hbm, src_hbm, dst_hbm, zeros_hbm, zerosv_hbm, *refs):
        if with_cnt:
            out_hbm, cnt_hbm = refs[0], refs[1]
            refs = refs[2:]
        else:
            out_hbm = refs[0]
            cnt_v = None
            refs = refs[1:]
        is_v, id_v, rows_a, rows_b = refs[0:4]
        refs = refs[4:]
        if with_cnt:
            cnt_v = refs[0]
            refs = refs[1:]
        acc_sh, gsa, gsb, ssa, ssb = refs
        cid = lax.axis_index("c")
        sid = lax.axis_index("s")
        wid = sid * 2 + cid
        r0 = sid * SUBROWS
        pltpu.sync_copy(zeros_hbm.at[pl.ds(r0, SUBROWS)],
                        acc_sh.at[pl.ds(r0, SUBROWS)])
        if with_cnt:
            pltpu.sync_copy(zerosv_hbm, cnt_v)
        plsc.subcore_barrier()
        ones16 = jnp.ones((16,), jnp.float32)
        c0 = wid * chunks

        def g_issue(q, buf, sem):
            pltpu.async_copy(table_hbm.at[is_v.at[q]], buf, sem)

        def g_drain(q, buf, sem):
            pltpu.make_async_copy(table_hbm.at[is_v.at[q]], buf, sem).wait()

        def cnt_add(q):
            if with_cnt:
                for v in range(cw // 16):
                    plsc.addupdate_scatter(
                        cnt_v, [id_v[q, pl.ds(v * 16, 16)]], ones16)

        def blk_body(blk, carry):
            b0 = c0 + blk * ib
            pltpu.sync_copy(src_hbm.at[pl.ds(b0, ib)], is_v)
            pltpu.sync_copy(dst_hbm.at[pl.ds(b0, ib)], id_v)
            g_issue(0, rows_a, gsa)
            g_issue(1, rows_b, gsb)

            def pair(p, carry2):
                qa = 2 * p
                qb = 2 * p + 1
                g_drain(qa, rows_a, gsa)
                da = pltpu.async_copy(rows_a, acc_sh.at[id_v.at[qa]], ssa,
                                      add=True)
                cnt_add(qa)
                da.wait()

                @pl.when(qa + 2 < ib)
                def _():
                    g_issue(qa + 2, rows_a, gsa)

                g_drain(qb, rows_b, gsb)
                db = pltpu.async_copy(rows_b, acc_sh.at[id_v.at[qb]], ssb,
                                      add=True)
                cnt_add(qb)
                db.wait()

                @pl.when(qb + 2 < ib)
                def _():
                    g_issue(qb + 2, rows_b, gsb)

                return carry2

            lax.fori_loop(0, ib // 2, pair, 0)
            return carry

        lax.fori_loop(0, nblk, blk_body, 0)
        plsc.subcore_barrier()
        pltpu.sync_copy(acc_sh.at[pl.ds(r0, SUBROWS)],
                        out_hbm.at[cid, pl.ds(r0, SUBROWS)])
        if with_cnt:
            pltpu.sync_copy(cnt_v, cnt_hbm.at[wid])

    return k(table, src_r, dst_r, zeros, zerosv)


def _prep_edges(edge_index, cw):
    src = edge_index[0].astype(jnp.int32)
    dst = edge_index[1].astype(jnp.int32)
    pad = EPAD - E
    src_p = jnp.concatenate([src, jnp.zeros((pad,), jnp.int32)])
    dst_p = jnp.concatenate([dst, jnp.full((pad,), N, jnp.int32)])
    return src_p.reshape(EPAD // cw, cw), dst_p.reshape(EPAD // cw, cw)


def _dotT(a, w):
    return lax.dot_general(a, w, (((1,), (1,)), ((), ())),
                           preferred_element_type=jnp.float32)


def _layer1_block(p_ref, cp_ref, x_ref, wl_ref, bl_ref, wr_ref, g_ref, b_ref,
                  wl2_ref, wr2_ref, xp_ref, xr_ref, cnt_ref):
    agg = p_ref[0] + p_ref[1]
    cnt = jnp.sum(cp_ref[...], axis=-1)
    cntc = jnp.maximum(cnt, 1.0)
    mean = agg / cntc[:, None]
    h = _dotT(mean, wl_ref[...]) + _dotT(x_ref[...], wr_ref[...]) + bl_ref[...]
    h = jnp.maximum(h, 0.0)
    mu = jnp.mean(h, axis=1, keepdims=True)
    var = jnp.mean((h - mu) ** 2, axis=1, keepdims=True)
    hn = (h - mu) * lax.rsqrt(var + 1e-5) * g_ref[...] + b_ref[...]
    xp_ref[...] = _dotT(hn, wl2_ref[...])
    xr_ref[...] = _dotT(hn, wr2_ref[...])
    cnt_ref[...] = jnp.broadcast_to(cntc[:, None], (RB, 8))


def _layer1(P, CPT, x, Wl, bl, Wr, g, b, Wl2, Wr2):
    return pl.pallas_call(
        _layer1_block,
        grid=(N // RB,),
        in_specs=[
            pl.BlockSpec((2, RB, 128), lambda i: (0, i, 0)),
            pl.BlockSpec((RB, NW), lambda i: (i, 0)),
            pl.BlockSpec((RB, 128), lambda i: (i, 0)),
            pl.BlockSpec((256, 128), lambda i: (0, 0)),
            pl.BlockSpec((256,), lambda i: (0,)),
            pl.BlockSpec((256, 128), lambda i: (0, 0)),
            pl.BlockSpec((256,), lambda i: (0,)),
            pl.BlockSpec((256,), lambda i: (0,)),
            pl.BlockSpec((128, 256), lambda i: (0, 0)),
            pl.BlockSpec((128, 256), lambda i: (0, 0)),
        ],
        out_specs=[
            pl.BlockSpec((RB, 128), lambda i: (i, 0)),
            pl.BlockSpec((RB, 128), lambda i: (i, 0)),
            pl.BlockSpec((RB, 8), lambda i: (i, 0)),
        ],
        out_shape=[
            jax.ShapeDtypeStruct((N, 128), jnp.float32),
            jax.ShapeDtypeStruct((N, 128), jnp.float32),
            jax.ShapeDtypeStruct((N, 8), jnp.float32),
        ],
    )(P, CPT, x, Wl, bl, Wr, g, b, Wl2, Wr2)


def _layer2_block(q_ref, cnt_ref, xr_ref, bl_ref, g_ref, b_ref,
                  w1_ref, b1_ref, w2_ref, b2_ref, w3_ref, b3_ref, o_ref):
    rinv = 1.0 / cnt_ref[:, :1]
    h = (q_ref[0] + q_ref[1]) * rinv + xr_ref[...] + bl_ref[...]
    h = jnp.maximum(h, 0.0)
    mu = jnp.mean(h, axis=1, keepdims=True)
    var = jnp.mean((h - mu) ** 2, axis=1, keepdims=True)
    hn = (h - mu) * lax.rsqrt(var + 1e-5) * g_ref[...] + b_ref[...]
    z = jnp.maximum(_dotT(hn, w1_ref[...]) + b1_ref[...], 0.0)
    z = jnp.maximum(_dotT(z, w2_ref[...]) + b2_ref[...], 0.0)
    z = jnp.maximum(_dotT(z, w3_ref[...]) + b3_ref[...], 0.0)
    o_ref[...] = z


def _layer2(Q, cnt, Xr, bl, g, b, W1, b1, W2, b2, W3, b3):
    full = lambda r, c: pl.BlockSpec((r, c), lambda i: (0, 0))
    vec = lambda r: pl.BlockSpec((r,), lambda i: (0,))
    return pl.pallas_call(
        _layer2_block,
        grid=(N // RB,),
        in_specs=[
            pl.BlockSpec((2, RB, 128), lambda i: (0, i, 0)),
            pl.BlockSpec((RB, 8), lambda i: (i, 0)),
            pl.BlockSpec((RB, 128), lambda i: (i, 0)),
            vec(128), vec(128), vec(128),
            full(256, 128), vec(256), full(128, 256), vec(128),
            full(64, 128), vec(64),
        ],
        out_specs=pl.BlockSpec((RB, 64), lambda i: (i, 0)),
        out_shape=jax.ShapeDtypeStruct((N, 64), jnp.float32),
    )(Q, cnt, Xr, bl, g, b, W1, b1, W2, b2, W3, b3)


def _final_block(x_ref, y_ref, o_ref):
    o_ref[...] = lax.dot_general(x_ref[...], y_ref[...],
                                 (((1,), (1,)), ((), ())),
                                 preferred_element_type=jnp.float32)


def _final_matmul(x, y):
    RBF = 400
    return pl.pallas_call(
        _final_block,
        grid=(N // RBF,),
        in_specs=[pl.BlockSpec((RBF, 64), lambda i: (i, 0)),
                  pl.BlockSpec((N, 64), lambda i: (0, 0))],
        out_specs=pl.BlockSpec((RBF, N), lambda i: (i, 0)),
        out_shape=jax.ShapeDtypeStruct((N, N), jnp.float32),
    )(x, y)


def _branch(x, edge_index, Wl1, bl1, Wr1, g1, b1, Wl2, bl2, Wr2, g2, b2,
            W1, c1, W2, c2, W3, c3):
    src_r, dst_r = _prep_edges(edge_index, CW)
    src_r2, dst_r2 = _prep_edges(edge_index, CW2)
    zeros128 = jnp.zeros((NPAD, 128), jnp.float32)
    zerosv = jnp.zeros((NPAD,), jnp.float32)
    P, CNTP = _sc_segsum(x, src_r, dst_r, zeros128, zerosv, True, CW)
    Xp, Xr, cnt = _layer1(P, CNTP.T, x, Wl1, bl1, Wr1, g1, b1, Wl2, Wr2)
    (Q,) = _sc_segsum(Xp, src_r2, dst_r2, zeros128, zerosv, False, CW2)
    return _layer2(Q, cnt, Xr, bl2, g2, b2, W1, c1, W2, c2, W3, c3)


def kernel(mm_edge_index, dd_edge_index, x_m, x_d,
           Wl_x1, bl_x1, Wr_x1, g_x1, b_x1,
           Wl_x2, bl_x2, Wr_x2, g_x2, b_x2,
           Wl_y1, bl_y1, Wr_y1, g_y1, b_y1,
           Wl_y2, bl_y2, Wr_y2, g_y2, b_y2,
           W1x, b1x, W2x, b2x, W3x, b3x,
           W1y, b1y, W2y, b2y, W3y, b3y):
    xk = _branch(x_m, mm_edge_index, Wl_x1, bl_x1, Wr_x1, g_x1, b_x1,
                 Wl_x2, bl_x2, Wr_x2, g_x2, b_x2,
                 W1x, b1x, W2x, b2x, W3x, b3x)
    yk = _branch(x_d, dd_edge_index, Wl_y1, bl_y1, Wr_y1, g_y1, b_y1,
                 Wl_y2, bl_y2, Wr_y2, g_y2, b_y2,
                 W1y, b1y, W2y, b2y, W3y, b3y)
    return _final_matmul(xk, yk)
